# baseline (speedup 1.0000x reference)
import functools

import jax
import jax.numpy as jnp
from jax import lax
from jax.experimental import pallas as pl
from jax.experimental.pallas import tpu as pltpu
from jax.experimental.pallas import tpu_sc as plsc

VOCAB = 1000000
EMB = 16
BATCH = 16384
SEQ = 200
NW = 32
ROWS_W = BATCH // NW
CHUNK_ROWS = 128
N_CHUNKS = ROWS_W // CHUNK_ROWS
CHUNK_IDS = CHUNK_ROWS * SEQ
DT_BLK = 65536


def _dtable_body(embt_ref, w_ref, out_ref):
    out_ref[...] = jnp.sum(embt_ref[...] * w_ref[...], axis=0)


def _make_dtable(embt, wcol):
    import math
    grid = math.ceil(VOCAB / DT_BLK)
    return pl.pallas_call(
        _dtable_body,
        grid=(grid,),
        in_specs=[pl.BlockSpec((EMB, DT_BLK), lambda i: (0, i)),
                  pl.BlockSpec((EMB, 1), lambda i: (0, 0))],
        out_specs=pl.BlockSpec((DT_BLK,), lambda i: (i,)),
        out_shape=jax.ShapeDtypeStruct((VOCAB,), jnp.float32),
    )(embt, wcol)


def _rg_body(ids_ref, out_ref):
    out_ref[...] = ids_ref[...].reshape(CHUNK_IDS)


def _regroup_ids(ids_t):
    g = BATCH // CHUNK_ROWS
    return pl.pallas_call(
        _rg_body,
        grid=(g,),
        in_specs=[pl.BlockSpec((SEQ, CHUNK_ROWS), lambda i: (0, i))],
        out_specs=pl.BlockSpec((CHUNK_IDS,), lambda i: (i,)),
        out_shape=jax.ShapeDtypeStruct((BATCH * SEQ,), jnp.int32),
    )(ids_t)


def _sc_body(ids_hbm, dt_hbm, consts_hbm, out0_hbm, out1_hbm,
             idx_a, idx_b, g_a, g_b, out0_v, out1_v, consts_v,
             sem_a, sem_b, ssem_a, ssem_b):
    c = lax.axis_index("c")
    s = lax.axis_index("s")
    wid = s * 2 + c
    row_base = wid * ROWS_W

    pltpu.sync_copy(consts_hbm, consts_v)
    corr = consts_v[...]

    bufs = [(idx_a, g_a, sem_a, ssem_a), (idx_b, g_b, sem_b, ssem_b)]

    def stage(k, idx_v, ssem):
        pltpu.async_copy(
            ids_hbm.at[pl.ds((wid * N_CHUNKS + k) * CHUNK_IDS, CHUNK_IDS)],
            idx_v, ssem).wait()

    def gather_start(idx_v, g_v, sem):
        pltpu.async_copy(dt_hbm.at[idx_v], g_v, sem)

    def gather_wait(idx_v, g_v, sem):
        pltpu.make_async_copy(dt_hbm.at[idx_v], g_v, sem).wait()

    def compute(k, g_v):
        for j in range(8):
            def tok_step(t, acc, j=j):
                return acc + g_v[pl.ds(t * CHUNK_ROWS + j * 16, 16)]
            z = lax.fori_loop(1, SEQ, tok_step, g_v[pl.ds(j * 16, 16)])
            z = z + corr
            p0 = 1.0 / (1.0 + jnp.exp(-z))
            out0_v[pl.ds(k * CHUNK_ROWS + j * 16, 16)] = p0
            out1_v[pl.ds(k * CHUNK_ROWS + j * 16, 16)] = 1.0 - p0

    for k in range(2):
        idx_v, g_v, sem, ssem = bufs[k]
        stage(k, idx_v, ssem)
        gather_start(idx_v, g_v, sem)
    for k in range(N_CHUNKS):
        idx_v, g_v, sem, ssem = bufs[k % 2]
        gather_wait(idx_v, g_v, sem)
        compute(k, g_v)
        if k + 2 < N_CHUNKS:
            stage(k + 2, idx_v, ssem)
            gather_start(idx_v, g_v, sem)

    pltpu.sync_copy(out0_v, out0_hbm.at[pl.ds(row_base, ROWS_W)])
    pltpu.sync_copy(out1_v, out1_hbm.at[pl.ds(row_base, ROWS_W)])


@functools.partial(
    pl.kernel,
    mesh=plsc.VectorSubcoreMesh(core_axis_name="c", subcore_axis_name="s"),
    out_type=(jax.ShapeDtypeStruct((BATCH,), jnp.float32),
              jax.ShapeDtypeStruct((BATCH,), jnp.float32)),
    scratch_types=[
        pltpu.VMEM((CHUNK_IDS,), jnp.int32),
        pltpu.VMEM((CHUNK_IDS,), jnp.int32),
        pltpu.VMEM((CHUNK_IDS,), jnp.float32),
        pltpu.VMEM((CHUNK_IDS,), jnp.float32),
        pltpu.VMEM((ROWS_W,), jnp.float32),
        pltpu.VMEM((ROWS_W,), jnp.float32),
        pltpu.VMEM((16,), jnp.float32),
        pltpu.SemaphoreType.DMA,
        pltpu.SemaphoreType.DMA,
        pltpu.SemaphoreType.DMA,
        pltpu.SemaphoreType.DMA,
    ],
)
def _sc_kernel(ids_hbm, dt_hbm, consts_hbm, out0_hbm, out1_hbm, *scratch):
    _sc_body(ids_hbm, dt_hbm, consts_hbm, out0_hbm, out1_hbm, *scratch)


def kernel(input_ids, emb_table, W, b):
    wdiff = (W[0] - W[1]) * (1.0 / SEQ)
    dtable = _make_dtable(emb_table.T, wdiff[:, None])

    ids_rg = _regroup_ids(input_ids.astype(jnp.int32).T)

    consts = jnp.full((16,), b[0] - b[1], jnp.float32)
    p0, p1 = _sc_kernel(ids_rg, dtable, consts)
    return jnp.stack([p0, p1], axis=1)


# speedup vs baseline: 1.3954x; 1.3954x over previous
import functools

import jax
import jax.numpy as jnp
from jax import lax
from jax.experimental import pallas as pl
from jax.experimental.pallas import tpu as pltpu
from jax.experimental.pallas import tpu_sc as plsc

VOCAB = 1000000
EMB = 16
BATCH = 16384
SEQ = 200
NW = 32
ROWS_W = BATCH // NW
CHUNK_ROWS = 128
N_CHUNKS = ROWS_W // CHUNK_ROWS
CHUNK_IDS = CHUNK_ROWS * SEQ
DT_BLK = 131072


def _dtable_body(embt_ref, w_ref, out_ref):
    out_ref[...] = jnp.sum(embt_ref[...] * w_ref[...], axis=0)


def _make_dtable(embt, wcol):
    import math
    grid = math.ceil(VOCAB / DT_BLK)
    return pl.pallas_call(
        _dtable_body,
        grid=(grid,),
        in_specs=[pl.BlockSpec((EMB, DT_BLK), lambda i: (0, i)),
                  pl.BlockSpec((EMB, 1), lambda i: (0, 0))],
        out_specs=pl.BlockSpec((DT_BLK,), lambda i: (i,)),
        out_shape=jax.ShapeDtypeStruct((VOCAB,), jnp.float32),
    )(embt, wcol)


def _sc_body(ids_hbm, dt_hbm, consts_hbm, out0_hbm, out1_hbm,
             idx_a, idx_b, g_a, g_b, out0_v, out1_v, consts_v,
             sem_a, sem_b, ssem_a, ssem_b):
    c = lax.axis_index("c")
    s = lax.axis_index("s")
    wid = s * 2 + c
    row_base = wid * ROWS_W

    pltpu.sync_copy(consts_hbm, consts_v)
    corr = consts_v[...]

    bufs = [(idx_a, g_a, sem_a, ssem_a), (idx_b, g_b, sem_b, ssem_b)]

    def stage(k, idx_v, ssem):
        base = row_base + k * CHUNK_ROWS

        def issue(q, carry2):
            for u in range(4):
                t = q * 4 + u
                pltpu.async_copy(
                    ids_hbm.at[t, pl.ds(base, CHUNK_ROWS)],
                    idx_v.at[pl.ds(t * CHUNK_ROWS, CHUNK_ROWS)], ssem)
            return carry2
        lax.fori_loop(0, SEQ // 4, issue, 0)

        def drain(q, carry2):
            for u in range(4):
                t = q * 4 + u
                pltpu.make_async_copy(
                    ids_hbm.at[t, pl.ds(base, CHUNK_ROWS)],
                    idx_v.at[pl.ds(t * CHUNK_ROWS, CHUNK_ROWS)], ssem).wait()
            return carry2
        lax.fori_loop(0, SEQ // 4, drain, 0)

    def gather_start(idx_v, g_v, sem):
        pltpu.async_copy(dt_hbm.at[idx_v], g_v, sem)

    def gather_wait(idx_v, g_v, sem):
        pltpu.make_async_copy(dt_hbm.at[idx_v], g_v, sem).wait()

    def compute(k, g_v):
        for j in range(8):
            def tok_step(t, acc, j=j):
                return acc + g_v[pl.ds(t * CHUNK_ROWS + j * 16, 16)]
            z = lax.fori_loop(1, SEQ, tok_step, g_v[pl.ds(j * 16, 16)])
            z = z + corr
            p0 = 1.0 / (1.0 + jnp.exp(-z))
            out0_v[pl.ds(k * CHUNK_ROWS + j * 16, 16)] = p0
            out1_v[pl.ds(k * CHUNK_ROWS + j * 16, 16)] = 1.0 - p0

    for k in range(2):
        idx_v, g_v, sem, ssem = bufs[k]
        stage(k, idx_v, ssem)
        gather_start(idx_v, g_v, sem)
    for k in range(N_CHUNKS):
        idx_v, g_v, sem, ssem = bufs[k % 2]
        gather_wait(idx_v, g_v, sem)
        compute(k, g_v)
        if k + 2 < N_CHUNKS:
            stage(k + 2, idx_v, ssem)
            gather_start(idx_v, g_v, sem)

    pltpu.sync_copy(out0_v, out0_hbm.at[pl.ds(row_base, ROWS_W)])
    pltpu.sync_copy(out1_v, out1_hbm.at[pl.ds(row_base, ROWS_W)])


@functools.partial(
    pl.kernel,
    mesh=plsc.VectorSubcoreMesh(core_axis_name="c", subcore_axis_name="s"),
    out_type=(jax.ShapeDtypeStruct((BATCH,), jnp.float32),
              jax.ShapeDtypeStruct((BATCH,), jnp.float32)),
    scratch_types=[
        pltpu.VMEM((CHUNK_IDS,), jnp.int32),
        pltpu.VMEM((CHUNK_IDS,), jnp.int32),
        pltpu.VMEM((CHUNK_IDS,), jnp.float32),
        pltpu.VMEM((CHUNK_IDS,), jnp.float32),
        pltpu.VMEM((ROWS_W,), jnp.float32),
        pltpu.VMEM((ROWS_W,), jnp.float32),
        pltpu.VMEM((16,), jnp.float32),
        pltpu.SemaphoreType.DMA,
        pltpu.SemaphoreType.DMA,
        pltpu.SemaphoreType.DMA,
        pltpu.SemaphoreType.DMA,
    ],
)
def _sc_kernel(ids_hbm, dt_hbm, consts_hbm, out0_hbm, out1_hbm, *scratch):
    _sc_body(ids_hbm, dt_hbm, consts_hbm, out0_hbm, out1_hbm, *scratch)


def kernel(input_ids, emb_table, W, b):
    wdiff = (W[0] - W[1]) * (1.0 / SEQ)
    dtable = _make_dtable(emb_table.T, wdiff[:, None])

    ids_t = input_ids.astype(jnp.int32).T

    consts = jnp.full((16,), b[0] - b[1], jnp.float32)
    p0, p1 = _sc_kernel(ids_t, dtable, consts)
    return jnp.stack([p0, p1], axis=1)
